# Initial kernel scaffold; baseline (speedup 1.0000x reference)
import jax
import jax.numpy as jnp
from jax.experimental import pallas as pl


def kernel(x, table):
    raise NotImplementedError("write your pallas kernel here")


# speedup vs baseline: 4.5361x; 4.5361x over previous
import functools

import jax
import jax.numpy as jnp
from jax import lax
from jax.experimental import pallas as pl
from jax.experimental.pallas import tpu as pltpu
from jax.experimental.pallas import tpu_sc as plsc

NC, NS = 2, 16
NW = NC * NS
CHUNK = 128
GPB = 8
BLOCK = CHUNK * GPB


def _sc_gather(x2d, table, B, D):
    b_per_w = B // NW
    nblocks = b_per_w // BLOCK
    mesh = plsc.VectorSubcoreMesh(core_axis_name="c", subcore_axis_name="s")

    @functools.partial(
        pl.kernel,
        out_type=jax.ShapeDtypeStruct((B, D), jnp.float32),
        mesh=mesh,
        scratch_types=[
            pltpu.VMEM((GPB, CHUNK), jnp.int32),
            pltpu.VMEM((BLOCK, D), jnp.float32),
            pltpu.SemaphoreType.DMA,
        ],
        compiler_params=pltpu.CompilerParams(use_tc_tiling_on_sc=False),
    )
    def k(x_hbm, table_hbm, out_hbm, idx_v, rows_v, sem):
        wid = lax.axis_index("s") * NC + lax.axis_index("c")
        row0 = wid * (b_per_w // CHUNK)

        def body(g, carry):
            pltpu.sync_copy(x_hbm.at[pl.ds(row0 + g * GPB, GPB)], idx_v)
            copies = [
                pltpu.async_copy(
                    table_hbm.at[idx_v.at[j]],
                    rows_v.at[pl.ds(j * CHUNK, CHUNK)],
                    sem,
                )
                for j in range(GPB)
            ]
            for c in copies:
                c.wait()
            pltpu.sync_copy(
                rows_v, out_hbm.at[pl.ds(wid * b_per_w + g * BLOCK, BLOCK)]
            )
            return carry

        lax.fori_loop(0, nblocks, body, 0)

    return k(x2d, table)


def kernel(x, table):
    S0, S1 = x.shape
    D = table.shape[1]
    B = S0 * S1
    x2d = x.reshape(B // CHUNK, CHUNK).astype(jnp.int32)
    out = _sc_gather(x2d, table, B, D)
    return out.reshape(S0, S1, D)


# speedup vs baseline: 4.5549x; 1.0042x over previous
import functools

import jax
import jax.numpy as jnp
from jax import lax
from jax.experimental import pallas as pl
from jax.experimental.pallas import tpu as pltpu
from jax.experimental.pallas import tpu_sc as plsc

NC, NS = 2, 16
NW = NC * NS
CHUNK = 128
GPB = 8
BLOCK = CHUNK * GPB
NIDX = 4
NROW = 2


def _sc_gather(x2d, table, B, D):
    b_per_w = B // NW
    nblocks = b_per_w // BLOCK
    mesh = plsc.VectorSubcoreMesh(core_axis_name="c", subcore_axis_name="s")

    @functools.partial(
        pl.kernel,
        out_type=jax.ShapeDtypeStruct((B, D), jnp.float32),
        mesh=mesh,
        scratch_types=[
            pltpu.VMEM((NIDX, GPB, CHUNK), jnp.int32),
            pltpu.VMEM((NROW, BLOCK, D), jnp.float32),
            [pltpu.SemaphoreType.DMA] * NIDX,
            [pltpu.SemaphoreType.DMA] * NROW,
            [pltpu.SemaphoreType.DMA] * NROW,
        ],
        compiler_params=pltpu.CompilerParams(use_tc_tiling_on_sc=False),
    )
    def k(x_hbm, table_hbm, out_hbm, idx_v, rows_v, sem_idx, sem_g, sem_out):
        wid = lax.axis_index("s") * NC + lax.axis_index("c")
        row0 = wid * (b_per_w // CHUNK)
        obase = wid * b_per_w

        def start_idx(g, k_slot):
            @pl.when(g < nblocks)
            def _():
                pltpu.async_copy(
                    x_hbm.at[pl.ds(row0 + g * GPB, GPB)],
                    idx_v.at[k_slot],
                    sem_idx[k_slot],
                )

        def wait_idx(k_slot):
            pltpu.make_async_copy(
                x_hbm.at[pl.ds(row0, GPB)], idx_v.at[k_slot], sem_idx[k_slot]
            ).wait()

        def wait_out(s):
            pltpu.make_async_copy(
                rows_v.at[s], out_hbm.at[pl.ds(obase, BLOCK)], sem_out[s]
            ).wait()

        def block_step(g, k_slot, first_round):
            s = k_slot % NROW
            if not first_round or k_slot >= NROW:
                wait_out(s)
            wait_idx(k_slot)
            copies = [
                pltpu.async_copy(
                    table_hbm.at[idx_v.at[k_slot, j]],
                    rows_v.at[s, pl.ds(j * CHUNK, CHUNK)],
                    sem_g[s],
                )
                for j in range(GPB)
            ]
            for c in copies:
                c.wait()
            pltpu.async_copy(
                rows_v.at[s],
                out_hbm.at[pl.ds(obase + g * BLOCK, BLOCK)],
                sem_out[s],
            )
            start_idx(g + NIDX, k_slot)

        for k_slot in range(NIDX):
            start_idx(jnp.int32(k_slot), k_slot)
        for k_slot in range(NIDX):
            block_step(jnp.int32(k_slot), k_slot, first_round=True)

        def body(i, carry):
            g0 = i * NIDX
            for k_slot in range(NIDX):
                block_step(g0 + k_slot, k_slot, first_round=False)
            return carry

        lax.fori_loop(1, nblocks // NIDX, body, 0)
        for s in range(NROW):
            wait_out(s)

    return k(x2d, table)


def kernel(x, table):
    S0, S1 = x.shape
    D = table.shape[1]
    B = S0 * S1
    x2d = x.reshape(B // CHUNK, CHUNK).astype(jnp.int32)
    out = _sc_gather(x2d, table, B, D)
    return out.reshape(S0, S1, D)
